# baseline (speedup 1.0000x reference)
import jax
import jax.numpy as jnp
from jax.experimental import pallas as pl
from jax.experimental.pallas import tpu as pltpu

_B, _S, _D, _H = 1, 2048, 768, 12
_EA, _EF, _K = 8, 8, 2
_DH = _D // _H
_DFF = 4 * _D
_SB_C = 512
_SB_F = 256
_QB = 128


def _mm_t(a, b):
    return jax.lax.dot_general(a, b, (((1,), (1,)), ((), ())),
                               preferred_element_type=jnp.float32)


def _router_a_body(x_ref, mrw_ref, mrb_ref, ar_ref, ti_ref, awk_ref):
    xm = jnp.mean(x_ref[...], axis=0, keepdims=True)
    logits = jnp.dot(xm, mrw_ref[...],
                     preferred_element_type=jnp.float32) + mrb_ref[...]
    ar = jax.nn.softmax(logits, axis=-1)
    iota = jax.lax.broadcasted_iota(jnp.int32, (1, _EA), 1)
    i1 = jnp.argmax(ar, axis=-1)
    m1 = iota == i1[:, None]
    i2 = jnp.argmax(jnp.where(m1, -1.0, ar), axis=-1)
    m2 = iota == i2[:, None]
    mask = (m1 | m2).astype(jnp.float32)
    aw = ar * mask
    aw = aw / (jnp.sum(aw, axis=-1, keepdims=True) + 1e-9)
    w1 = jnp.sum(jnp.where(m1, aw, 0.0), axis=-1)
    w2 = jnp.sum(jnp.where(m2, aw, 0.0), axis=-1)
    ar_ref[...] = ar
    ti_ref[...] = jnp.stack([i1, i2], axis=-1).astype(jnp.int32)
    awk_ref[...] = jnp.stack([w1, w2], axis=-1)


def _qkv_body(ti_ref, x_ref, w_ref, b_ref, out_ref):
    del ti_ref
    y = _mm_t(x_ref[...], w_ref[0]) + b_ref[0]
    out_ref[0] = y


def _attn_body(q_ref, k_ref, v_ref, o_ref):
    outs = []
    for hh in range(2):
        q = q_ref[0][:, hh * _DH:(hh + 1) * _DH]
        k = k_ref[0][:, hh * _DH:(hh + 1) * _DH]
        v = v_ref[0][:, hh * _DH:(hh + 1) * _DH]
        s = _mm_t(q, k) * (1.0 / 8.0)
        p = jax.nn.softmax(s, axis=-1)
        outs.append(jnp.dot(p, v, preferred_element_type=jnp.float32))
    o_ref[0] = jnp.concatenate(outs, axis=1)


def _combine_body(ti_ref, o_ref, wo_ref, bo_ref, wp_ref, bp_ref, awk_ref,
                  x_ref, g1_ref, b1_ref, out_ref, acc_ref):
    del ti_ref
    e = pl.program_id(0)
    sb = pl.program_id(1)
    t = _mm_t(o_ref[0], wo_ref[0]) + bo_ref[0]
    r = _mm_t(t, wp_ref[0]) + bp_ref[0]
    w_e = jnp.sum(jnp.where(
        jax.lax.broadcasted_iota(jnp.int32, (1, _K), 1) == e,
        awk_ref[...], 0.0))
    c = w_e * r
    sl = pl.ds(sb * _SB_C, _SB_C)

    @pl.when(e == 0)
    def _():
        acc_ref[sl, :] = c

    @pl.when(e == 1)
    def _():
        attn = acc_ref[sl, :] + c
        out_ref[...] = (g1_ref[...] * (x_ref[...] + attn)
                        / jnp.sqrt(1.0 + 1e-5) + b1_ref[...])


def _router_f_body(x1_ref, frw_ref, frb_ref, fr_ref, fw_ref):
    logits = jnp.dot(x1_ref[...], frw_ref[...],
                     preferred_element_type=jnp.float32) + frb_ref[...]
    fr = jax.nn.softmax(logits, axis=-1)
    iota = jax.lax.broadcasted_iota(jnp.int32, (_S, _EF), 1)
    i1 = jnp.argmax(fr, axis=-1)
    m1 = iota == i1[:, None]
    i2 = jnp.argmax(jnp.where(m1, -1.0, fr), axis=-1)
    m2 = iota == i2[:, None]
    mask = (m1 | m2).astype(jnp.float32)
    fw = fr * mask
    fw = fw / (jnp.sum(fw, axis=-1, keepdims=True) + 1e-9)
    fr_ref[...] = fr
    fw_ref[...] = fw


def _ffn_body(x1_ref, w1_ref, b1_ref, w2_ref, b2_ref, fw_ref, g2_ref,
              bn2_ref, out_ref, acc_ref):
    e = pl.program_id(0)
    sb = pl.program_id(1)
    xb = x1_ref[...].astype(jnp.bfloat16)
    z = _mm_t(xb, w1_ref[0]) + b1_ref[0]
    h = 0.5 * z * (1.0 + jax.lax.erf(z * (2.0 ** -0.5)))
    y = _mm_t(h.astype(jnp.bfloat16), w2_ref[0]) + b2_ref[0]
    sel = (jax.lax.broadcasted_iota(jnp.int32, (_SB_F, _EF), 1) == e)
    wcol = jnp.sum(jnp.where(sel, fw_ref[...], 0.0), axis=-1,
                   keepdims=True)
    c = y * wcol
    sl = pl.ds(sb * _SB_F, _SB_F)

    @pl.when(e == 0)
    def _():
        acc_ref[sl, :] = c

    @pl.when(e > 0)
    def _():
        acc_ref[sl, :] = acc_ref[sl, :] + c

    @pl.when(e == _EF - 1)
    def _():
        out_ref[...] = (g2_ref[...] * (x1_ref[...] + acc_ref[sl, :])
                        / jnp.sqrt(1.0 + 1e-5) + bn2_ref[...])


def kernel(x, qkv_w, qkv_b, mha_out_w, mha_out_b, proj_w, proj_b, mr_w, mr_b,
           fr_w, fr_b, fc1_w, fc1_b, fc2_w, fc2_b, g1, b1, g2, b2):
    f32 = jnp.float32
    x2d = x.reshape(_S, _D)
    mrb = mr_b.reshape(1, _EA)
    frb = fr_b.reshape(1, _EF)
    g1r, b1r = g1.reshape(1, _D), b1.reshape(1, _D)
    g2r, b2r = g2.reshape(1, _D), b2.reshape(1, _D)

    ar, ti, awk = pl.pallas_call(
        _router_a_body,
        out_shape=(jax.ShapeDtypeStruct((1, _EA), f32),
                   jax.ShapeDtypeStruct((1, _K), jnp.int32),
                   jax.ShapeDtypeStruct((1, _K), f32)),
    )(x2d, mr_w, mrb)
    ti1 = ti.reshape(_K)

    qkv = pl.pallas_call(
        _qkv_body,
        grid_spec=pltpu.PrefetchScalarGridSpec(
            num_scalar_prefetch=1,
            grid=(_K, 3),
            in_specs=[
                pl.BlockSpec((_S, _D), lambda e, nb, ti: (0, 0)),
                pl.BlockSpec((1, _D, _D), lambda e, nb, ti: (ti[e], nb, 0)),
                pl.BlockSpec((1, 1, _D), lambda e, nb, ti: (ti[e], 0, nb)),
            ],
            out_specs=pl.BlockSpec((1, _S, _D), lambda e, nb, ti: (e, 0, nb)),
        ),
        out_shape=jax.ShapeDtypeStruct((_K, _S, 3 * _D), f32),
    )(ti1, x2d, qkv_w, qkv_b.reshape(_EA, 1, 3 * _D))

    _HP = _H // 2
    o = pl.pallas_call(
        _attn_body,
        grid=(_K, _HP, _S // _QB),
        in_specs=[
            pl.BlockSpec((1, _QB, 2 * _DH), lambda e, hp, qb: (e, qb, hp)),
            pl.BlockSpec((1, _S, 2 * _DH), lambda e, hp, qb: (e, 0, _HP + hp)),
            pl.BlockSpec((1, _S, 2 * _DH),
                         lambda e, hp, qb: (e, 0, 2 * _HP + hp)),
        ],
        out_specs=pl.BlockSpec((1, _QB, 2 * _DH), lambda e, hp, qb: (e, qb, hp)),
        out_shape=jax.ShapeDtypeStruct((_K, _S, _D), f32),
    )(qkv, qkv, qkv)

    x1 = pl.pallas_call(
        _combine_body,
        grid_spec=pltpu.PrefetchScalarGridSpec(
            num_scalar_prefetch=1,
            grid=(_K, _S // _SB_C),
            in_specs=[
                pl.BlockSpec((1, _SB_C, _D), lambda e, sb, ti: (e, sb, 0)),
                pl.BlockSpec((1, _D, _D), lambda e, sb, ti: (ti[e], 0, 0)),
                pl.BlockSpec((1, 1, _D), lambda e, sb, ti: (ti[e], 0, 0)),
                pl.BlockSpec((1, _D, _D), lambda e, sb, ti: (ti[e], 0, 0)),
                pl.BlockSpec((1, 1, _D), lambda e, sb, ti: (ti[e], 0, 0)),
                pl.BlockSpec((1, _K), lambda e, sb, ti: (0, 0)),
                pl.BlockSpec((_SB_C, _D), lambda e, sb, ti: (sb, 0)),
                pl.BlockSpec((1, _D), lambda e, sb, ti: (0, 0)),
                pl.BlockSpec((1, _D), lambda e, sb, ti: (0, 0)),
            ],
            out_specs=pl.BlockSpec((_SB_C, _D), lambda e, sb, ti: (sb, 0)),
            scratch_shapes=[pltpu.VMEM((_S, _D), f32)],
        ),
        out_shape=jax.ShapeDtypeStruct((_S, _D), f32),
    )(ti1, o, mha_out_w, mha_out_b.reshape(_EA, 1, _D), proj_w,
      proj_b.reshape(_EA, 1, _D), awk, x2d, g1r, b1r)

    fr, fw = pl.pallas_call(
        _router_f_body,
        out_shape=(jax.ShapeDtypeStruct((_S, _EF), f32),
                   jax.ShapeDtypeStruct((_S, _EF), f32)),
    )(x1, fr_w, frb)

    x2 = pl.pallas_call(
        _ffn_body,
        grid=(_EF, _S // _SB_F),
        in_specs=[
            pl.BlockSpec((_SB_F, _D), lambda e, sb: (sb, 0)),
            pl.BlockSpec((1, _DFF, _D), lambda e, sb: (e, 0, 0)),
            pl.BlockSpec((1, 1, _DFF), lambda e, sb: (e, 0, 0)),
            pl.BlockSpec((1, _D, _DFF), lambda e, sb: (e, 0, 0)),
            pl.BlockSpec((1, 1, _D), lambda e, sb: (e, 0, 0)),
            pl.BlockSpec((_SB_F, _EF), lambda e, sb: (sb, 0)),
            pl.BlockSpec((1, _D), lambda e, sb: (0, 0)),
            pl.BlockSpec((1, _D), lambda e, sb: (0, 0)),
        ],
        out_specs=pl.BlockSpec((_SB_F, _D), lambda e, sb: (sb, 0)),
        out_shape=jax.ShapeDtypeStruct((_S, _D), f32),
        scratch_shapes=[pltpu.VMEM((_S, _D), f32)],
    )(x1, fc1_w.astype(jnp.bfloat16), fc1_b.reshape(_EF, 1, _DFF),
      fc2_w.astype(jnp.bfloat16), fc2_b.reshape(_EF, 1, _D), fw, g2r, b2r)

    return (x2.reshape(_B, _S, _D), ar.reshape(_EA), fr)


# speedup vs baseline: 1.0902x; 1.0902x over previous
import jax
import jax.numpy as jnp
from jax.experimental import pallas as pl
from jax.experimental.pallas import tpu as pltpu

_B, _S, _D, _H = 1, 2048, 768, 12
_EA, _EF, _K = 8, 8, 2
_DH = _D // _H
_DFF = 4 * _D
_SB_C = 512
_SB_F = 256
_QB = 128


def _mm_t(a, b):
    return jax.lax.dot_general(a, b, (((1,), (1,)), ((), ())),
                               preferred_element_type=jnp.float32)


def _router_a_body(x_ref, mrw_ref, mrb_ref, ar_ref, ti_ref, awk_ref):
    xm = jnp.mean(x_ref[...], axis=0, keepdims=True)
    logits = jnp.dot(xm, mrw_ref[...],
                     preferred_element_type=jnp.float32) + mrb_ref[...]
    ar = jax.nn.softmax(logits, axis=-1)
    iota = jax.lax.broadcasted_iota(jnp.int32, (1, _EA), 1)
    i1 = jnp.argmax(ar, axis=-1)
    m1 = iota == i1[:, None]
    i2 = jnp.argmax(jnp.where(m1, -1.0, ar), axis=-1)
    m2 = iota == i2[:, None]
    mask = (m1 | m2).astype(jnp.float32)
    aw = ar * mask
    aw = aw / (jnp.sum(aw, axis=-1, keepdims=True) + 1e-9)
    w1 = jnp.sum(jnp.where(m1, aw, 0.0), axis=-1)
    w2 = jnp.sum(jnp.where(m2, aw, 0.0), axis=-1)
    ar_ref[...] = ar
    ti_ref[...] = jnp.stack([i1, i2], axis=-1).astype(jnp.int32)
    awk_ref[...] = jnp.stack([w1, w2], axis=-1)


def _qkv_body(ti_ref, x_ref, w_ref, b_ref, out_ref):
    del ti_ref
    y = _mm_t(x_ref[...], w_ref[0]) + b_ref[0]
    out_ref[0] = y


def _attn_body(q_ref, k_ref, v_ref, o_ref):
    outs = []
    for hh in range(2):
        q = q_ref[0][:, hh * _DH:(hh + 1) * _DH]
        k = k_ref[0][:, hh * _DH:(hh + 1) * _DH]
        v = v_ref[0][:, hh * _DH:(hh + 1) * _DH]
        s = _mm_t(q, k) * (1.0 / 8.0)
        p = jax.nn.softmax(s, axis=-1)
        outs.append(jnp.dot(p, v, preferred_element_type=jnp.float32))
    o_ref[0] = jnp.concatenate(outs, axis=1)


def _combine_body(ti_ref, o_ref, wo_ref, bo_ref, wp_ref, bp_ref, awk_ref,
                  x_ref, g1_ref, b1_ref, out_ref, acc_ref):
    del ti_ref
    e = pl.program_id(0)
    sb = pl.program_id(1)
    t = _mm_t(o_ref[0], wo_ref[0]) + bo_ref[0]
    r = _mm_t(t, wp_ref[0]) + bp_ref[0]
    w_e = jnp.sum(jnp.where(
        jax.lax.broadcasted_iota(jnp.int32, (1, _K), 1) == e,
        awk_ref[...], 0.0))
    c = w_e * r
    sl = pl.ds(sb * _SB_C, _SB_C)

    @pl.when(e == 0)
    def _():
        acc_ref[sl, :] = c

    @pl.when(e == 1)
    def _():
        attn = acc_ref[sl, :] + c
        out_ref[...] = (g1_ref[...] * (x_ref[...] + attn)
                        / jnp.sqrt(1.0 + 1e-5) + b1_ref[...])


def _router_f_body(x1_ref, frw_ref, frb_ref, fr_ref, fw_ref):
    logits = jnp.dot(x1_ref[...], frw_ref[...],
                     preferred_element_type=jnp.float32) + frb_ref[...]
    fr = jax.nn.softmax(logits, axis=-1)
    iota = jax.lax.broadcasted_iota(jnp.int32, (_S, _EF), 1)
    i1 = jnp.argmax(fr, axis=-1)
    m1 = iota == i1[:, None]
    i2 = jnp.argmax(jnp.where(m1, -1.0, fr), axis=-1)
    m2 = iota == i2[:, None]
    mask = (m1 | m2).astype(jnp.float32)
    fw = fr * mask
    fw = fw / (jnp.sum(fw, axis=-1, keepdims=True) + 1e-9)
    fr_ref[...] = fr
    fw_ref[...] = fw


def _ffn_body(x1_ref, w1_ref, b1_ref, w2_ref, b2_ref, fw_ref, g2_ref,
              bn2_ref, out_ref, acc_ref):
    e = pl.program_id(0)
    sb = pl.program_id(1)
    z = _mm_t(x1_ref[...], w1_ref[0]) + b1_ref[0]
    h = 0.5 * z * (1.0 + jax.lax.erf(z * (2.0 ** -0.5)))
    y = _mm_t(h, w2_ref[0]) + b2_ref[0]
    sel = (jax.lax.broadcasted_iota(jnp.int32, (_SB_F, _EF), 1) == e)
    wcol = jnp.sum(jnp.where(sel, fw_ref[...], 0.0), axis=-1,
                   keepdims=True)
    c = y * wcol
    sl = pl.ds(sb * _SB_F, _SB_F)

    @pl.when(e == 0)
    def _():
        acc_ref[sl, :] = c

    @pl.when(e > 0)
    def _():
        acc_ref[sl, :] = acc_ref[sl, :] + c

    @pl.when(e == _EF - 1)
    def _():
        out_ref[...] = (g2_ref[...] * (x1_ref[...] + acc_ref[sl, :])
                        / jnp.sqrt(1.0 + 1e-5) + bn2_ref[...])


def kernel(x, qkv_w, qkv_b, mha_out_w, mha_out_b, proj_w, proj_b, mr_w, mr_b,
           fr_w, fr_b, fc1_w, fc1_b, fc2_w, fc2_b, g1, b1, g2, b2):
    f32 = jnp.float32
    x2d = x.reshape(_S, _D)
    mrb = mr_b.reshape(1, _EA)
    frb = fr_b.reshape(1, _EF)
    g1r, b1r = g1.reshape(1, _D), b1.reshape(1, _D)
    g2r, b2r = g2.reshape(1, _D), b2.reshape(1, _D)

    ar, ti, awk = pl.pallas_call(
        _router_a_body,
        out_shape=(jax.ShapeDtypeStruct((1, _EA), f32),
                   jax.ShapeDtypeStruct((1, _K), jnp.int32),
                   jax.ShapeDtypeStruct((1, _K), f32)),
    )(x2d, mr_w, mrb)
    ti1 = ti.reshape(_K)

    qkv = pl.pallas_call(
        _qkv_body,
        grid_spec=pltpu.PrefetchScalarGridSpec(
            num_scalar_prefetch=1,
            grid=(_K, 3),
            in_specs=[
                pl.BlockSpec((_S, _D), lambda e, nb, ti: (0, 0)),
                pl.BlockSpec((1, _D, _D), lambda e, nb, ti: (ti[e], nb, 0)),
                pl.BlockSpec((1, 1, _D), lambda e, nb, ti: (ti[e], 0, nb)),
            ],
            out_specs=pl.BlockSpec((1, _S, _D), lambda e, nb, ti: (e, 0, nb)),
        ),
        out_shape=jax.ShapeDtypeStruct((_K, _S, 3 * _D), f32),
    )(ti1, x2d, qkv_w, qkv_b.reshape(_EA, 1, 3 * _D))

    _HP = _H // 2
    o = pl.pallas_call(
        _attn_body,
        grid=(_K, _HP, _S // _QB),
        in_specs=[
            pl.BlockSpec((1, _QB, 2 * _DH), lambda e, hp, qb: (e, qb, hp)),
            pl.BlockSpec((1, _S, 2 * _DH), lambda e, hp, qb: (e, 0, _HP + hp)),
            pl.BlockSpec((1, _S, 2 * _DH),
                         lambda e, hp, qb: (e, 0, 2 * _HP + hp)),
        ],
        out_specs=pl.BlockSpec((1, _QB, 2 * _DH), lambda e, hp, qb: (e, qb, hp)),
        out_shape=jax.ShapeDtypeStruct((_K, _S, _D), f32),
    )(qkv, qkv, qkv)

    x1 = pl.pallas_call(
        _combine_body,
        grid_spec=pltpu.PrefetchScalarGridSpec(
            num_scalar_prefetch=1,
            grid=(_K, _S // _SB_C),
            in_specs=[
                pl.BlockSpec((1, _SB_C, _D), lambda e, sb, ti: (e, sb, 0)),
                pl.BlockSpec((1, _D, _D), lambda e, sb, ti: (ti[e], 0, 0)),
                pl.BlockSpec((1, 1, _D), lambda e, sb, ti: (ti[e], 0, 0)),
                pl.BlockSpec((1, _D, _D), lambda e, sb, ti: (ti[e], 0, 0)),
                pl.BlockSpec((1, 1, _D), lambda e, sb, ti: (ti[e], 0, 0)),
                pl.BlockSpec((1, _K), lambda e, sb, ti: (0, 0)),
                pl.BlockSpec((_SB_C, _D), lambda e, sb, ti: (sb, 0)),
                pl.BlockSpec((1, _D), lambda e, sb, ti: (0, 0)),
                pl.BlockSpec((1, _D), lambda e, sb, ti: (0, 0)),
            ],
            out_specs=pl.BlockSpec((_SB_C, _D), lambda e, sb, ti: (sb, 0)),
            scratch_shapes=[pltpu.VMEM((_S, _D), f32)],
        ),
        out_shape=jax.ShapeDtypeStruct((_S, _D), f32),
    )(ti1, o, mha_out_w, mha_out_b.reshape(_EA, 1, _D), proj_w,
      proj_b.reshape(_EA, 1, _D), awk, x2d, g1r, b1r)

    fr, fw = pl.pallas_call(
        _router_f_body,
        out_shape=(jax.ShapeDtypeStruct((_S, _EF), f32),
                   jax.ShapeDtypeStruct((_S, _EF), f32)),
    )(x1, fr_w, frb)

    x2 = pl.pallas_call(
        _ffn_body,
        grid=(_EF, _S // _SB_F),
        in_specs=[
            pl.BlockSpec((_SB_F, _D), lambda e, sb: (sb, 0)),
            pl.BlockSpec((1, _DFF, _D), lambda e, sb: (e, 0, 0)),
            pl.BlockSpec((1, 1, _DFF), lambda e, sb: (e, 0, 0)),
            pl.BlockSpec((1, _D, _DFF), lambda e, sb: (e, 0, 0)),
            pl.BlockSpec((1, 1, _D), lambda e, sb: (e, 0, 0)),
            pl.BlockSpec((_SB_F, _EF), lambda e, sb: (sb, 0)),
            pl.BlockSpec((1, _D), lambda e, sb: (0, 0)),
            pl.BlockSpec((1, _D), lambda e, sb: (0, 0)),
        ],
        out_specs=pl.BlockSpec((_SB_F, _D), lambda e, sb: (sb, 0)),
        out_shape=jax.ShapeDtypeStruct((_S, _D), f32),
        scratch_shapes=[pltpu.VMEM((_S, _D), f32)],
    )(x1, fc1_w, fc1_b.reshape(_EF, 1, _DFF), fc2_w,
      fc2_b.reshape(_EF, 1, _D), fw, g2r, b2r)

    return (x2.reshape(_B, _S, _D), ar.reshape(_EA), fr)


# speedup vs baseline: 1.3325x; 1.2223x over previous
import jax
import jax.numpy as jnp
from jax.experimental import pallas as pl
from jax.experimental.pallas import tpu as pltpu

_B, _S, _D, _H = 1, 2048, 768, 12
_EA, _EF, _K = 8, 8, 2
_DH = _D // _H
_DFF = 4 * _D
_SB_C = 512
_SB_F = 256
_QB = 256


def _mm_t(a, b):
    return jax.lax.dot_general(a, b, (((1,), (1,)), ((), ())),
                               preferred_element_type=jnp.float32)


def _router_a_body(x_ref, mrw_ref, mrb_ref, ar_ref, ti_ref, awk_ref):
    xm = jnp.mean(x_ref[...], axis=0, keepdims=True)
    logits = jnp.dot(xm, mrw_ref[...],
                     preferred_element_type=jnp.float32) + mrb_ref[...]
    ar = jax.nn.softmax(logits, axis=-1)
    iota = jax.lax.broadcasted_iota(jnp.int32, (1, _EA), 1)
    i1 = jnp.argmax(ar, axis=-1)
    m1 = iota == i1[:, None]
    i2 = jnp.argmax(jnp.where(m1, -1.0, ar), axis=-1)
    m2 = iota == i2[:, None]
    mask = (m1 | m2).astype(jnp.float32)
    aw = ar * mask
    aw = aw / (jnp.sum(aw, axis=-1, keepdims=True) + 1e-9)
    w1 = jnp.sum(jnp.where(m1, aw, 0.0), axis=-1)
    w2 = jnp.sum(jnp.where(m2, aw, 0.0), axis=-1)
    ar_ref[...] = ar
    ti_ref[...] = jnp.stack([i1, i2], axis=-1).astype(jnp.int32)
    awk_ref[...] = jnp.stack([w1, w2], axis=-1)


def _qkv_body(ti_ref, x_ref, w_ref, b_ref, out_ref):
    del ti_ref
    y = _mm_t(x_ref[...], w_ref[0]) + b_ref[0]
    out_ref[0] = y


def _attn_body(q_ref, k_ref, v_ref, o_ref):
    log2e_over_sqrt_dh = 1.4426950408889634 / 8.0
    outs = []
    for hh in range(2):
        q = q_ref[0][:, hh * _DH:(hh + 1) * _DH] * log2e_over_sqrt_dh
        k = k_ref[0][:, hh * _DH:(hh + 1) * _DH]
        v = v_ref[0][:, hh * _DH:(hh + 1) * _DH]
        s2 = _mm_t(q, k)
        m = jnp.max(s2, axis=-1, keepdims=True)
        e2 = jnp.exp2(s2 - m)
        u = jnp.dot(e2, v, preferred_element_type=jnp.float32)
        r = 1.0 / jnp.sum(e2, axis=-1, keepdims=True)
        outs.append(u * r)
    o_ref[0] = jnp.concatenate(outs, axis=1)


def _combine_body(ti_ref, o_ref, wo_ref, bo_ref, wp_ref, bp_ref, awk_ref,
                  x_ref, g1_ref, b1_ref, out_ref, acc_ref):
    del ti_ref
    e = pl.program_id(0)
    sb = pl.program_id(1)
    t = _mm_t(o_ref[0], wo_ref[0]) + bo_ref[0]
    r = _mm_t(t, wp_ref[0]) + bp_ref[0]
    w_e = jnp.sum(jnp.where(
        jax.lax.broadcasted_iota(jnp.int32, (1, _K), 1) == e,
        awk_ref[...], 0.0))
    c = w_e * r
    sl = pl.ds(sb * _SB_C, _SB_C)

    @pl.when(e == 0)
    def _():
        acc_ref[sl, :] = c

    @pl.when(e == 1)
    def _():
        attn = acc_ref[sl, :] + c
        out_ref[...] = (g1_ref[...] * (x_ref[...] + attn)
                        / jnp.sqrt(1.0 + 1e-5) + b1_ref[...])


def _router_f_body(x1_ref, frw_ref, frb_ref, fr_ref, fw_ref):
    logits = jnp.dot(x1_ref[...], frw_ref[...],
                     preferred_element_type=jnp.float32) + frb_ref[...]
    fr = jax.nn.softmax(logits, axis=-1)
    iota = jax.lax.broadcasted_iota(jnp.int32, (_S, _EF), 1)
    i1 = jnp.argmax(fr, axis=-1)
    m1 = iota == i1[:, None]
    i2 = jnp.argmax(jnp.where(m1, -1.0, fr), axis=-1)
    m2 = iota == i2[:, None]
    mask = (m1 | m2).astype(jnp.float32)
    fw = fr * mask
    fw = fw / (jnp.sum(fw, axis=-1, keepdims=True) + 1e-9)
    fr_ref[...] = fr
    fw_ref[...] = fw


def _ffn_body(x1_ref, w1_ref, b1_ref, w2_ref, b2_ref, fw_ref, g2_ref,
              bn2_ref, out_ref, acc_ref):
    e = pl.program_id(0)
    sb = pl.program_id(1)
    z = _mm_t(x1_ref[...], w1_ref[0]) + b1_ref[0]
    h = 0.5 * z * (1.0 + jax.lax.erf(z * (2.0 ** -0.5)))
    y = _mm_t(h, w2_ref[0]) + b2_ref[0]
    sel = (jax.lax.broadcasted_iota(jnp.int32, (_SB_F, _EF), 1) == e)
    wcol = jnp.sum(jnp.where(sel, fw_ref[...], 0.0), axis=-1,
                   keepdims=True)
    c = y * wcol
    sl = pl.ds(sb * _SB_F, _SB_F)

    @pl.when(e == 0)
    def _():
        acc_ref[sl, :] = c

    @pl.when(e > 0)
    def _():
        acc_ref[sl, :] = acc_ref[sl, :] + c

    @pl.when(e == _EF - 1)
    def _():
        out_ref[...] = (g2_ref[...] * (x1_ref[...] + acc_ref[sl, :])
                        / jnp.sqrt(1.0 + 1e-5) + bn2_ref[...])


def kernel(x, qkv_w, qkv_b, mha_out_w, mha_out_b, proj_w, proj_b, mr_w, mr_b,
           fr_w, fr_b, fc1_w, fc1_b, fc2_w, fc2_b, g1, b1, g2, b2):
    f32 = jnp.float32
    x2d = x.reshape(_S, _D)
    mrb = mr_b.reshape(1, _EA)
    frb = fr_b.reshape(1, _EF)
    g1r, b1r = g1.reshape(1, _D), b1.reshape(1, _D)
    g2r, b2r = g2.reshape(1, _D), b2.reshape(1, _D)

    ar, ti, awk = pl.pallas_call(
        _router_a_body,
        out_shape=(jax.ShapeDtypeStruct((1, _EA), f32),
                   jax.ShapeDtypeStruct((1, _K), jnp.int32),
                   jax.ShapeDtypeStruct((1, _K), f32)),
    )(x2d, mr_w, mrb)
    ti1 = ti.reshape(_K)

    qkv = pl.pallas_call(
        _qkv_body,
        grid_spec=pltpu.PrefetchScalarGridSpec(
            num_scalar_prefetch=1,
            grid=(_K, 3),
            in_specs=[
                pl.BlockSpec((_S, _D), lambda e, nb, ti: (0, 0)),
                pl.BlockSpec((1, _D, _D), lambda e, nb, ti: (ti[e], nb, 0)),
                pl.BlockSpec((1, 1, _D), lambda e, nb, ti: (ti[e], 0, nb)),
            ],
            out_specs=pl.BlockSpec((1, _S, _D), lambda e, nb, ti: (e, 0, nb)),
        ),
        out_shape=jax.ShapeDtypeStruct((_K, _S, 3 * _D), f32),
    )(ti1, x2d, qkv_w, qkv_b.reshape(_EA, 1, 3 * _D))

    _HP = _H // 2
    o = pl.pallas_call(
        _attn_body,
        grid=(_K, _HP, _S // _QB),
        in_specs=[
            pl.BlockSpec((1, _QB, 2 * _DH), lambda e, hp, qb: (e, qb, hp)),
            pl.BlockSpec((1, _S, 2 * _DH), lambda e, hp, qb: (e, 0, _HP + hp)),
            pl.BlockSpec((1, _S, 2 * _DH),
                         lambda e, hp, qb: (e, 0, 2 * _HP + hp)),
        ],
        out_specs=pl.BlockSpec((1, _QB, 2 * _DH), lambda e, hp, qb: (e, qb, hp)),
        out_shape=jax.ShapeDtypeStruct((_K, _S, _D), f32),
    )(qkv, qkv, qkv)

    x1 = pl.pallas_call(
        _combine_body,
        grid_spec=pltpu.PrefetchScalarGridSpec(
            num_scalar_prefetch=1,
            grid=(_K, _S // _SB_C),
            in_specs=[
                pl.BlockSpec((1, _SB_C, _D), lambda e, sb, ti: (e, sb, 0)),
                pl.BlockSpec((1, _D, _D), lambda e, sb, ti: (ti[e], 0, 0)),
                pl.BlockSpec((1, 1, _D), lambda e, sb, ti: (ti[e], 0, 0)),
                pl.BlockSpec((1, _D, _D), lambda e, sb, ti: (ti[e], 0, 0)),
                pl.BlockSpec((1, 1, _D), lambda e, sb, ti: (ti[e], 0, 0)),
                pl.BlockSpec((1, _K), lambda e, sb, ti: (0, 0)),
                pl.BlockSpec((_SB_C, _D), lambda e, sb, ti: (sb, 0)),
                pl.BlockSpec((1, _D), lambda e, sb, ti: (0, 0)),
                pl.BlockSpec((1, _D), lambda e, sb, ti: (0, 0)),
            ],
            out_specs=pl.BlockSpec((_SB_C, _D), lambda e, sb, ti: (sb, 0)),
            scratch_shapes=[pltpu.VMEM((_S, _D), f32)],
        ),
        out_shape=jax.ShapeDtypeStruct((_S, _D), f32),
    )(ti1, o, mha_out_w, mha_out_b.reshape(_EA, 1, _D), proj_w,
      proj_b.reshape(_EA, 1, _D), awk, x2d, g1r, b1r)

    fr, fw = pl.pallas_call(
        _router_f_body,
        out_shape=(jax.ShapeDtypeStruct((_S, _EF), f32),
                   jax.ShapeDtypeStruct((_S, _EF), f32)),
    )(x1, fr_w, frb)

    x2 = pl.pallas_call(
        _ffn_body,
        grid=(_EF, _S // _SB_F),
        in_specs=[
            pl.BlockSpec((_SB_F, _D), lambda e, sb: (sb, 0)),
            pl.BlockSpec((1, _DFF, _D), lambda e, sb: (e, 0, 0)),
            pl.BlockSpec((1, 1, _DFF), lambda e, sb: (e, 0, 0)),
            pl.BlockSpec((1, _D, _DFF), lambda e, sb: (e, 0, 0)),
            pl.BlockSpec((1, 1, _D), lambda e, sb: (e, 0, 0)),
            pl.BlockSpec((_SB_F, _EF), lambda e, sb: (sb, 0)),
            pl.BlockSpec((1, _D), lambda e, sb: (0, 0)),
            pl.BlockSpec((1, _D), lambda e, sb: (0, 0)),
        ],
        out_specs=pl.BlockSpec((_SB_F, _D), lambda e, sb: (sb, 0)),
        out_shape=jax.ShapeDtypeStruct((_S, _D), f32),
        scratch_shapes=[pltpu.VMEM((_S, _D), f32)],
    )(x1, fc1_w, fc1_b.reshape(_EF, 1, _DFF), fc2_w,
      fc2_b.reshape(_EF, 1, _D), fw, g2r, b2r)

    return (x2.reshape(_B, _S, _D), ar.reshape(_EA), fr)


# speedup vs baseline: 1.3400x; 1.0056x over previous
import jax
import jax.numpy as jnp
from jax.experimental import pallas as pl
from jax.experimental.pallas import tpu as pltpu

_B, _S, _D, _H = 1, 2048, 768, 12
_EA, _EF, _K = 8, 8, 2
_DH = _D // _H
_DFF = 4 * _D
_SB_C = 512
_SB_F = 256
_QB = 256


def _mm_t(a, b):
    return jax.lax.dot_general(a, b, (((1,), (1,)), ((), ())),
                               preferred_element_type=jnp.float32)


def _router_a_body(x_ref, mrw_ref, mrb_ref, ar_ref, ti_ref, awk_ref):
    xm = jnp.mean(x_ref[...], axis=0, keepdims=True)
    logits = jnp.dot(xm, mrw_ref[...],
                     preferred_element_type=jnp.float32) + mrb_ref[...]
    ar = jax.nn.softmax(logits, axis=-1)
    iota = jax.lax.broadcasted_iota(jnp.int32, (1, _EA), 1)
    i1 = jnp.argmax(ar, axis=-1)
    m1 = iota == i1[:, None]
    i2 = jnp.argmax(jnp.where(m1, -1.0, ar), axis=-1)
    m2 = iota == i2[:, None]
    mask = (m1 | m2).astype(jnp.float32)
    aw = ar * mask
    aw = aw / (jnp.sum(aw, axis=-1, keepdims=True) + 1e-9)
    w1 = jnp.sum(jnp.where(m1, aw, 0.0), axis=-1)
    w2 = jnp.sum(jnp.where(m2, aw, 0.0), axis=-1)
    ar_ref[...] = ar
    ti_ref[...] = jnp.stack([i1, i2], axis=-1).astype(jnp.int32)
    awk_ref[...] = jnp.stack([w1, w2], axis=-1)


def _qkv_body(ti_ref, x_ref, w_ref, b_ref, out_ref):
    del ti_ref
    y = _mm_t(x_ref[...], w_ref[0]) + b_ref[0]
    out_ref[0] = y


def _attn_body(q_ref, k_ref, v_ref, o_ref):
    log2e_over_sqrt_dh = 1.4426950408889634 / 8.0
    outs = []
    for hh in range(2):
        q = q_ref[0][:, hh * _DH:(hh + 1) * _DH] * log2e_over_sqrt_dh
        k = k_ref[0][:, hh * _DH:(hh + 1) * _DH]
        v = v_ref[0][:, hh * _DH:(hh + 1) * _DH]
        s2 = _mm_t(q, k)
        m = jnp.max(s2, axis=-1, keepdims=True)
        e2 = jnp.exp2(s2 - m)
        u = jnp.dot(e2, v, preferred_element_type=jnp.float32)
        r = 1.0 / jnp.sum(e2, axis=-1, keepdims=True)
        outs.append(u * r)
    o_ref[0] = jnp.concatenate(outs, axis=1)


def _combine_body(ti_ref, o_ref, wo_ref, bo_ref, wp_ref, bp_ref, awk_ref,
                  x_ref, g1_ref, b1_ref, out_ref, acc_ref):
    del ti_ref
    e = pl.program_id(0)
    sb = pl.program_id(1)
    t = _mm_t(o_ref[0], wo_ref[0]) + bo_ref[0]
    r = _mm_t(t, wp_ref[0]) + bp_ref[0]
    w_e = jnp.sum(jnp.where(
        jax.lax.broadcasted_iota(jnp.int32, (1, _K), 1) == e,
        awk_ref[...], 0.0))
    c = w_e * r
    sl = pl.ds(sb * _SB_C, _SB_C)

    @pl.when(e == 0)
    def _():
        acc_ref[sl, :] = c

    @pl.when(e == 1)
    def _():
        attn = acc_ref[sl, :] + c
        out_ref[...] = (g1_ref[...] * (x_ref[...] + attn)
                        / jnp.sqrt(1.0 + 1e-5) + b1_ref[...])


def _ffn_body(x1_ref, w1_ref, b1_ref, w2_ref, b2_ref, frw_ref, frb_ref,
              g2_ref, bn2_ref, out_ref, fr_ref, acc_ref):
    e = pl.program_id(0)
    sb = pl.program_id(1)
    logits = jnp.dot(x1_ref[...], frw_ref[...],
                     preferred_element_type=jnp.float32) + frb_ref[...]
    fr = jax.nn.softmax(logits, axis=-1)
    iota = jax.lax.broadcasted_iota(jnp.int32, (_SB_F, _EF), 1)
    i1 = jnp.argmax(fr, axis=-1)
    m1 = iota == i1[:, None]
    i2 = jnp.argmax(jnp.where(m1, -1.0, fr), axis=-1)
    m2 = iota == i2[:, None]
    fwm = fr * (m1 | m2).astype(jnp.float32)
    fw = fwm / (jnp.sum(fwm, axis=-1, keepdims=True) + 1e-9)
    fr_ref[...] = fr

    z = _mm_t(x1_ref[...], w1_ref[0]) + b1_ref[0]
    h = 0.5 * z * (1.0 + jax.lax.erf(z * (2.0 ** -0.5)))
    y = _mm_t(h, w2_ref[0]) + b2_ref[0]
    wcol = jnp.sum(jnp.where(iota == e, fw, 0.0), axis=-1,
                   keepdims=True)
    c = y * wcol
    sl = pl.ds(sb * _SB_F, _SB_F)

    @pl.when(e == 0)
    def _():
        acc_ref[sl, :] = c

    @pl.when(e > 0)
    def _():
        acc_ref[sl, :] = acc_ref[sl, :] + c

    @pl.when(e == _EF - 1)
    def _():
        out_ref[...] = (g2_ref[...] * (x1_ref[...] + acc_ref[sl, :])
                        / jnp.sqrt(1.0 + 1e-5) + bn2_ref[...])


def kernel(x, qkv_w, qkv_b, mha_out_w, mha_out_b, proj_w, proj_b, mr_w, mr_b,
           fr_w, fr_b, fc1_w, fc1_b, fc2_w, fc2_b, g1, b1, g2, b2):
    f32 = jnp.float32
    x2d = x.reshape(_S, _D)
    mrb = mr_b.reshape(1, _EA)
    frb = fr_b.reshape(1, _EF)
    g1r, b1r = g1.reshape(1, _D), b1.reshape(1, _D)
    g2r, b2r = g2.reshape(1, _D), b2.reshape(1, _D)

    ar, ti, awk = pl.pallas_call(
        _router_a_body,
        out_shape=(jax.ShapeDtypeStruct((1, _EA), f32),
                   jax.ShapeDtypeStruct((1, _K), jnp.int32),
                   jax.ShapeDtypeStruct((1, _K), f32)),
    )(x2d, mr_w, mrb)
    ti1 = ti.reshape(_K)

    qkv = pl.pallas_call(
        _qkv_body,
        grid_spec=pltpu.PrefetchScalarGridSpec(
            num_scalar_prefetch=1,
            grid=(_K, 3),
            in_specs=[
                pl.BlockSpec((_S, _D), lambda e, nb, ti: (0, 0)),
                pl.BlockSpec((1, _D, _D), lambda e, nb, ti: (ti[e], nb, 0)),
                pl.BlockSpec((1, 1, _D), lambda e, nb, ti: (ti[e], 0, nb)),
            ],
            out_specs=pl.BlockSpec((1, _S, _D), lambda e, nb, ti: (e, 0, nb)),
        ),
        out_shape=jax.ShapeDtypeStruct((_K, _S, 3 * _D), f32),
    )(ti1, x2d, qkv_w, qkv_b.reshape(_EA, 1, 3 * _D))

    _HP = _H // 2
    o = pl.pallas_call(
        _attn_body,
        grid=(_K, _HP, _S // _QB),
        in_specs=[
            pl.BlockSpec((1, _QB, 2 * _DH), lambda e, hp, qb: (e, qb, hp)),
            pl.BlockSpec((1, _S, 2 * _DH), lambda e, hp, qb: (e, 0, _HP + hp)),
            pl.BlockSpec((1, _S, 2 * _DH),
                         lambda e, hp, qb: (e, 0, 2 * _HP + hp)),
        ],
        out_specs=pl.BlockSpec((1, _QB, 2 * _DH), lambda e, hp, qb: (e, qb, hp)),
        out_shape=jax.ShapeDtypeStruct((_K, _S, _D), f32),
    )(qkv, qkv, qkv)

    x1 = pl.pallas_call(
        _combine_body,
        grid_spec=pltpu.PrefetchScalarGridSpec(
            num_scalar_prefetch=1,
            grid=(_K, _S // _SB_C),
            in_specs=[
                pl.BlockSpec((1, _SB_C, _D), lambda e, sb, ti: (e, sb, 0)),
                pl.BlockSpec((1, _D, _D), lambda e, sb, ti: (ti[e], 0, 0)),
                pl.BlockSpec((1, 1, _D), lambda e, sb, ti: (ti[e], 0, 0)),
                pl.BlockSpec((1, _D, _D), lambda e, sb, ti: (ti[e], 0, 0)),
                pl.BlockSpec((1, 1, _D), lambda e, sb, ti: (ti[e], 0, 0)),
                pl.BlockSpec((1, _K), lambda e, sb, ti: (0, 0)),
                pl.BlockSpec((_SB_C, _D), lambda e, sb, ti: (sb, 0)),
                pl.BlockSpec((1, _D), lambda e, sb, ti: (0, 0)),
                pl.BlockSpec((1, _D), lambda e, sb, ti: (0, 0)),
            ],
            out_specs=pl.BlockSpec((_SB_C, _D), lambda e, sb, ti: (sb, 0)),
            scratch_shapes=[pltpu.VMEM((_S, _D), f32)],
        ),
        out_shape=jax.ShapeDtypeStruct((_S, _D), f32),
    )(ti1, o, mha_out_w, mha_out_b.reshape(_EA, 1, _D), proj_w,
      proj_b.reshape(_EA, 1, _D), awk, x2d, g1r, b1r)

    x2, fr = pl.pallas_call(
        _ffn_body,
        grid=(_EF, _S // _SB_F),
        in_specs=[
            pl.BlockSpec((_SB_F, _D), lambda e, sb: (sb, 0)),
            pl.BlockSpec((1, _DFF, _D), lambda e, sb: (e, 0, 0)),
            pl.BlockSpec((1, 1, _DFF), lambda e, sb: (e, 0, 0)),
            pl.BlockSpec((1, _D, _DFF), lambda e, sb: (e, 0, 0)),
            pl.BlockSpec((1, 1, _D), lambda e, sb: (e, 0, 0)),
            pl.BlockSpec((_D, _EF), lambda e, sb: (0, 0)),
            pl.BlockSpec((1, _EF), lambda e, sb: (0, 0)),
            pl.BlockSpec((1, _D), lambda e, sb: (0, 0)),
            pl.BlockSpec((1, _D), lambda e, sb: (0, 0)),
        ],
        out_specs=[pl.BlockSpec((_SB_F, _D), lambda e, sb: (sb, 0)),
                   pl.BlockSpec((_SB_F, _EF), lambda e, sb: (sb, 0))],
        out_shape=(jax.ShapeDtypeStruct((_S, _D), f32),
                   jax.ShapeDtypeStruct((_S, _EF), f32)),
        scratch_shapes=[pltpu.VMEM((_S, _D), f32)],
    )(x1, fc1_w, fc1_b.reshape(_EF, 1, _DFF), fc2_w,
      fc2_b.reshape(_EF, 1, _D), fr_w, frb, g2r, b2r)

    return (x2.reshape(_B, _S, _D), ar.reshape(_EA), fr)
